# baseline (speedup 1.0000x reference)
import jax
import jax.numpy as jnp
from jax.experimental import pallas as pl
from jax.experimental.pallas import tpu as pltpu

_THRESHOLD = 0.95
_SECONDARY = 0.5

_G = 4
_NB = 8


_H = 512


def _copy(vbuf, out_hbm, sems, slot, step, g, h):
    rows = pl.ds(h * _H, _H)
    return pltpu.make_async_copy(
        vbuf.at[slot, g, rows], out_hbm.at[step * _G + g, rows],
        sems.at[slot, g, h])


def _lap_block(states_t_ref, out_hbm, vbuf, sems):
    b = pl.program_id(0)
    slot = jax.lax.rem(b, 2)

    @pl.when(b >= 2)
    def _wait_prev():
        for g in range(_G):
            for h in range(2):
                _copy(vbuf, out_hbm, sems, slot, b - 2, g, h).wait()

    for g in range(_G):
        st = states_t_ref[g]
        for h in range(2):
            srows = st[:, h * _H:(h + 1) * _H]
            gram = jax.lax.dot_general(
                srows, st, (((0,), (0,)), ((), ())),
                preferred_element_type=jnp.float32)
            fid = gram * gram
            wn = jnp.where(fid >= _THRESHOLD, jnp.float32(-1.0),
                           jnp.where(fid >= _SECONDARY, jnp.float32(-_SECONDARY),
                                     jnp.float32(0.0)))
            row = (jax.lax.broadcasted_iota(jnp.int32, wn.shape, 0) + h * _H)
            col = jax.lax.broadcasted_iota(jnp.int32, wn.shape, 1)
            diag = row == col
            wn = jnp.where(diag, jnp.float32(0.0), wn)
            deg = -jnp.sum(wn, axis=1, keepdims=True)
            vbuf[slot, g, h * _H:(h + 1) * _H] = jnp.where(diag, deg, wn)
            _copy(vbuf, out_hbm, sems, slot, b, g, h).start()

    @pl.when(b == _NB - 1)
    def _drain():
        for g in range(_G):
            for h in range(2):
                _copy(vbuf, out_hbm, sems, 1 - slot, b - 1, g, h).wait()
        for g in range(_G):
            for h in range(2):
                _copy(vbuf, out_hbm, sems, slot, b, g, h).wait()


def kernel(quantum_states):
    batch, num_states, n_wires = quantum_states.shape
    states_t = jnp.swapaxes(quantum_states, 1, 2)
    return pl.pallas_call(
        _lap_block,
        grid=(batch // _G,),
        in_specs=[
            pl.BlockSpec((_G, n_wires, num_states), lambda b: (b, 0, 0)),
        ],
        out_specs=pl.BlockSpec(memory_space=pltpu.MemorySpace.HBM),
        out_shape=jax.ShapeDtypeStruct((batch, num_states, num_states),
                                       jnp.float32),
        scratch_shapes=[
            pltpu.VMEM((2, _G, num_states, num_states), jnp.float32),
            pltpu.SemaphoreType.DMA((2, _G, 2)),
        ],
        compiler_params=pltpu.CompilerParams(
            dimension_semantics=("arbitrary",)),
    )(states_t)


# speedup vs baseline: 1.0057x; 1.0057x over previous
import jax
import jax.numpy as jnp
from jax.experimental import pallas as pl
from jax.experimental.pallas import tpu as pltpu

_THRESHOLD = 0.95
_SECONDARY = 0.5

_G = 4
_NB = 8


def _copy(vbuf, out_hbm, sems, slot, step, g):
    return pltpu.make_async_copy(
        vbuf.at[slot, g], out_hbm.at[step * _G + g], sems.at[slot, g])


def _lap_block(states_t_ref, out_hbm, vbuf, sems):
    b = pl.program_id(0)
    slot = jax.lax.rem(b, 2)

    @pl.when(b >= 2)
    def _wait_prev():
        for g in range(_G):
            _copy(vbuf, out_hbm, sems, slot, b - 2, g).wait()

    for g in range(_G):
        st = states_t_ref[g]
        gram = jax.lax.dot_general(
            st, st, (((0,), (0,)), ((), ())), preferred_element_type=jnp.float32)
        fid = gram * gram
        wn = jnp.where(fid >= _THRESHOLD, jnp.float32(-1.0),
                       jnp.where(fid >= _SECONDARY, jnp.float32(-_SECONDARY),
                                 jnp.float32(0.0)))
        row = jax.lax.broadcasted_iota(jnp.int32, wn.shape, 0)
        col = jax.lax.broadcasted_iota(jnp.int32, wn.shape, 1)
        diag = row == col
        wn = jnp.where(diag, jnp.float32(0.0), wn)
        deg = -jnp.sum(wn, axis=1, keepdims=True)
        vbuf[slot, g] = jnp.where(diag, deg, wn)
        _copy(vbuf, out_hbm, sems, slot, b, g).start()

    @pl.when(b == _NB - 1)
    def _drain():
        for g in range(_G):
            _copy(vbuf, out_hbm, sems, 1 - slot, b - 1, g).wait()
        for g in range(_G):
            _copy(vbuf, out_hbm, sems, slot, b, g).wait()


def kernel(quantum_states):
    batch, num_states, n_wires = quantum_states.shape
    states_t = jnp.swapaxes(quantum_states, 1, 2)
    return pl.pallas_call(
        _lap_block,
        grid=(batch // _G,),
        in_specs=[
            pl.BlockSpec((_G, n_wires, num_states), lambda b: (b, 0, 0)),
        ],
        out_specs=pl.BlockSpec(memory_space=pltpu.MemorySpace.HBM),
        out_shape=jax.ShapeDtypeStruct((batch, num_states, num_states),
                                       jnp.float32),
        scratch_shapes=[
            pltpu.VMEM((2, _G, num_states, num_states), jnp.float32),
            pltpu.SemaphoreType.DMA((2, _G)),
        ],
        compiler_params=pltpu.CompilerParams(
            dimension_semantics=("arbitrary",)),
    )(states_t)
